# baseline (speedup 1.0000x reference)
import functools

import jax
import jax.numpy as jnp
from jax import lax
from jax.experimental import pallas as pl
from jax.experimental.pallas import tpu as pltpu
from jax.experimental.pallas import tpu_sc as plsc

N = 10000
E = 320000
D = 128
ALPHA = 0.5

NC = 2
NS = 16
NP = 10112
RPT = NP // NS
EPW = E // NS
CHUNK = 128
KBUF = 2
IBLK = 40
NIB = 4
EPWP = NIB * IBLK * CHUNK
EPAD = EPWP - EPW
ZROWS = 8
KHIST = 8

_mesh = plsc.VectorSubcoreMesh(core_axis_name="c", subcore_axis_name="s")


@functools.partial(
    pl.kernel,
    out_type=(
        jax.ShapeDtypeStruct((NP, D), jnp.float32),
        jax.ShapeDtypeStruct((NP, D), jnp.float32),
    ),
    mesh=_mesh,
    scratch_types=[
        pltpu.VMEM((IBLK, CHUNK), jnp.int32),
        pltpu.VMEM((CHUNK, D), jnp.float32),
        pltpu.VMEM((ZROWS, D), jnp.float32),
        pltpu.VMEM_SHARED((NP, D), jnp.float32),
        pltpu.SemaphoreType.DMA,
    ],
)
def _deg_kernel(src_hbm, dst_hbm, dego_hbm, degi_hbm, idx_v, ones_v, zero_v, acc_sh, sem):
    c = lax.axis_index("c")
    s = lax.axis_index("s")

    one16 = jnp.full((16,), 1.0, jnp.float32)
    zero16 = jnp.zeros((16,), jnp.float32)

    @pl.loop(0, CHUNK)
    def _(i):
        @pl.loop(0, D // 16)
        def _(j):
            ones_v[i, pl.ds(j * 16, 16)] = one16

    @pl.loop(0, ZROWS)
    def _(i):
        @pl.loop(0, D // 16)
        def _(j):
            zero_v[i, pl.ds(j * 16, 16)] = zero16

    @pl.loop(0, RPT // ZROWS)
    def _(i):
        pltpu.sync_copy(zero_v, acc_sh.at[pl.ds(s * RPT + i * ZROWS, ZROWS)])

    plsc.subcore_barrier()

    def run_hist(idx_hbm, out_hbm):
        def drain_one():
            pltpu.make_async_copy(ones_v, acc_sh.at[idx_v.at[0]], sem).wait()

        @pl.loop(0, NIB)
        def _(p):
            pltpu.sync_copy(idx_hbm.at[s, p], idx_v)

            @pl.loop(0, IBLK)
            def _(j):
                pltpu.async_copy(ones_v, acc_sh.at[idx_v.at[j]], sem, add=True)

                @pl.when(j >= KHIST)
                def _():
                    drain_one()

            for _ in range(KHIST):
                drain_one()

        plsc.subcore_barrier()
        pltpu.sync_copy(
            acc_sh.at[pl.ds(s * RPT, RPT)], out_hbm.at[pl.ds(s * RPT, RPT)]
        )

    @pl.when(c == 0)
    def _():
        run_hist(src_hbm, dego_hbm)

    @pl.when(c == 1)
    def _():
        run_hist(dst_hbm, degi_hbm)


def _scale_body(x_ref, dego_ref, degi_ref, y_ref, z_ref):
    x = x_ref[...]
    dego = dego_ref[:N, 0:1]
    degi = degi_ref[:N, 0:1]
    dinv_out = jnp.where(dego > 0, lax.rsqrt(jnp.maximum(dego, 1e-12)), 0.0)
    dinv_in = jnp.where(degi > 0, lax.rsqrt(jnp.maximum(degi, 1e-12)), 0.0)
    y_ref[:N, :] = x * dinv_out
    z_ref[:N, :] = x * dinv_in
    pad = jnp.zeros((NP - N, D), jnp.float32)
    y_ref[N:, :] = pad
    z_ref[N:, :] = pad


_scale = pl.pallas_call(
    _scale_body,
    out_shape=(
        jax.ShapeDtypeStruct((NP, D), jnp.float32),
        jax.ShapeDtypeStruct((NP, D), jnp.float32),
    ),
)


@functools.partial(
    pl.kernel,
    out_type=(
        jax.ShapeDtypeStruct((NP, D), jnp.float32),
        jax.ShapeDtypeStruct((NP, D), jnp.float32),
    ),
    mesh=_mesh,
    scratch_types=[
        pltpu.VMEM((IBLK, CHUNK), jnp.int32),
        pltpu.VMEM((IBLK, CHUNK), jnp.int32),
        pltpu.VMEM((KBUF, CHUNK, D), jnp.float32),
        pltpu.VMEM_SHARED((NP, D), jnp.float32),
        [pltpu.SemaphoreType.DMA] * KBUF,
        [pltpu.SemaphoreType.DMA] * KBUF,
    ],
)
def _agg_kernel(
    y_hbm, z_hbm, src_hbm, dst_hbm, aggsd_hbm, aggds_hbm,
    gidx_v, sidx_v, rows_v, acc_sh, sem_g, sem_s,
):
    c = lax.axis_index("c")
    s = lax.axis_index("s")

    zero16 = jnp.zeros((16,), jnp.float32)

    @pl.loop(0, CHUNK)
    def _(i):
        @pl.loop(0, D // 16)
        def _(j):
            rows_v[0, i, pl.ds(j * 16, 16)] = zero16

    @pl.loop(0, 4)
    def _(i):
        pltpu.sync_copy(rows_v.at[0], acc_sh.at[pl.ds(s * RPT + i * CHUNK, CHUNK)])

    pltpu.sync_copy(
        rows_v.at[0, pl.ds(0, 120)], acc_sh.at[pl.ds(s * RPT + 4 * CHUNK, 120)]
    )

    plsc.subcore_barrier()

    def run_dir(tab_hbm, gidx_hbm, sidx_hbm, out_hbm):
        def gather_start(j, b):
            pltpu.async_copy(tab_hbm.at[gidx_v.at[j]], rows_v.at[b], sem_g[b])

        def gather_wait(j, b):
            pltpu.make_async_copy(tab_hbm.at[gidx_v.at[j]], rows_v.at[b], sem_g[b]).wait()

        def scatter_start(j, b):
            pltpu.async_copy(rows_v.at[b], acc_sh.at[sidx_v.at[j]], sem_s[b], add=True)

        def scatter_wait(b):
            pltpu.make_async_copy(rows_v.at[b], acc_sh.at[sidx_v.at[0]], sem_s[b]).wait()

        @pl.loop(0, NIB)
        def _(p):
            pltpu.sync_copy(gidx_hbm.at[s, p], gidx_v)
            pltpu.sync_copy(sidx_hbm.at[s, p], sidx_v)

            @pl.loop(0, IBLK // KBUF)
            def _(t):
                j0 = t * KBUF
                for b in range(KBUF):
                    @pl.when(t > 0)
                    def _(b=b):
                        scatter_wait(b)

                    gather_start(j0 + b, b)
                for b in range(KBUF):
                    gather_wait(j0 + b, b)
                    scatter_start(j0 + b, b)

            for b in range(KBUF):
                scatter_wait(b)

        plsc.subcore_barrier()
        pltpu.sync_copy(
            acc_sh.at[pl.ds(s * RPT, RPT)], out_hbm.at[pl.ds(s * RPT, RPT)]
        )

    @pl.when(c == 0)
    def _():
        run_dir(y_hbm, src_hbm, dst_hbm, aggsd_hbm)

    @pl.when(c == 1)
    def _():
        run_dir(z_hbm, dst_hbm, src_hbm, aggds_hbm)


def _combine_body(
    asd_ref, ads_ref, dego_ref, degi_ref, wsd_ref, wds_ref, bsd_ref, bds_ref, o_ref
):
    dego = dego_ref[:N, 0:1]
    degi = degi_ref[:N, 0:1]
    dinv_out = jnp.where(dego > 0, lax.rsqrt(jnp.maximum(dego, 1e-12)), 0.0)
    dinv_in = jnp.where(degi > 0, lax.rsqrt(jnp.maximum(degi, 1e-12)), 0.0)
    asd = asd_ref[:N, :] * dinv_in
    ads = ads_ref[:N, :] * dinv_out
    t_sd = (
        jnp.dot(
            asd,
            wsd_ref[...],
            preferred_element_type=jnp.float32,
            precision=lax.Precision.HIGHEST,
        )
        + bsd_ref[...]
    )
    t_ds = (
        jnp.dot(
            ads,
            wds_ref[...],
            preferred_element_type=jnp.float32,
            precision=lax.Precision.HIGHEST,
        )
        + bds_ref[...]
    )
    o_ref[...] = ALPHA * t_sd + (1.0 - ALPHA) * t_ds


_combine = pl.pallas_call(
    _combine_body,
    out_shape=jax.ShapeDtypeStruct((N, D), jnp.float32),
)


def kernel(x, edge_index, W_sd, b_sd, W_ds, b_ds):
    srcw = edge_index[0].reshape(NS, EPW)
    dstw = edge_index[1].reshape(NS, EPW)
    padv = jnp.full((NS, EPAD), NP - 1, dtype=jnp.int32)
    src4 = jnp.concatenate([srcw, padv], axis=1).reshape(NS, NIB, IBLK, CHUNK)
    dst4 = jnp.concatenate([dstw, padv], axis=1).reshape(NS, NIB, IBLK, CHUNK)

    dego, degi = _deg_kernel(src4, dst4)
    y, z = _scale(x, dego, degi)
    aggsd, aggds = _agg_kernel(y, z, src4, dst4)
    return _combine(
        aggsd,
        aggds,
        dego,
        degi,
        W_sd,
        W_ds,
        b_sd.reshape(1, D),
        b_ds.reshape(1, D),
    )


# speedup vs baseline: 1.5548x; 1.5548x over previous
import functools

import jax
import jax.numpy as jnp
from jax import lax
from jax.experimental import pallas as pl
from jax.experimental.pallas import tpu as pltpu
from jax.experimental.pallas import tpu_sc as plsc

N = 10000
E = 320000
D = 128
ALPHA = 0.5

NC = 2
NS = 16
NP = 10112
RPT = NP // NS
EPW = E // NS
CHUNK = 80
KBUF = 2
IBLK = 50
NIB = 5
ZROWS = 8
KHIST = 8

_mesh = plsc.VectorSubcoreMesh(core_axis_name="c", subcore_axis_name="s")


@functools.partial(
    pl.kernel,
    out_type=(
        jax.ShapeDtypeStruct((NP, D), jnp.float32),
        jax.ShapeDtypeStruct((NP, D), jnp.float32),
    ),
    mesh=_mesh,
    scratch_types=[
        pltpu.VMEM((IBLK, CHUNK), jnp.int32),
        pltpu.VMEM((CHUNK, D), jnp.float32),
        pltpu.VMEM((ZROWS, D), jnp.float32),
        pltpu.VMEM_SHARED((NP, D), jnp.float32),
        pltpu.SemaphoreType.DMA,
    ],
)
def _deg_kernel(src_hbm, dst_hbm, dego_hbm, degi_hbm, idx_v, ones_v, zero_v, acc_sh, sem):
    c = lax.axis_index("c")
    s = lax.axis_index("s")

    one16 = jnp.full((16,), 1.0, jnp.float32)
    zero16 = jnp.zeros((16,), jnp.float32)

    @pl.loop(0, CHUNK)
    def _(i):
        @pl.loop(0, D // 16)
        def _(j):
            ones_v[i, pl.ds(j * 16, 16)] = one16

    @pl.loop(0, ZROWS)
    def _(i):
        @pl.loop(0, D // 16)
        def _(j):
            zero_v[i, pl.ds(j * 16, 16)] = zero16

    @pl.loop(0, RPT // ZROWS)
    def _(i):
        pltpu.sync_copy(zero_v, acc_sh.at[pl.ds(s * RPT + i * ZROWS, ZROWS)])

    plsc.subcore_barrier()

    def run_hist(idx_hbm, out_hbm):
        def drain_one():
            pltpu.make_async_copy(ones_v, acc_sh.at[idx_v.at[0]], sem).wait()

        @pl.loop(0, NIB)
        def _(p):
            pltpu.sync_copy(idx_hbm.at[s, p], idx_v)

            @pl.loop(0, IBLK)
            def _(j):
                pltpu.async_copy(ones_v, acc_sh.at[idx_v.at[j]], sem, add=True)

                @pl.when(j >= KHIST)
                def _():
                    drain_one()

            for _ in range(KHIST):
                drain_one()

        plsc.subcore_barrier()
        pltpu.sync_copy(
            acc_sh.at[pl.ds(s * RPT, RPT)], out_hbm.at[pl.ds(s * RPT, RPT)]
        )

    @pl.when(c == 0)
    def _():
        run_hist(src_hbm, dego_hbm)

    @pl.when(c == 1)
    def _():
        run_hist(dst_hbm, degi_hbm)


def _scale_body(x_ref, dego_ref, degi_ref, y_ref, z_ref):
    x = x_ref[...]
    dego = dego_ref[:N, 0:1]
    degi = degi_ref[:N, 0:1]
    dinv_out = jnp.where(dego > 0, lax.rsqrt(jnp.maximum(dego, 1e-12)), 0.0)
    dinv_in = jnp.where(degi > 0, lax.rsqrt(jnp.maximum(degi, 1e-12)), 0.0)
    y_ref[:N, :] = x * dinv_out
    z_ref[:N, :] = x * dinv_in
    pad = jnp.zeros((NP - N, D), jnp.float32)
    y_ref[N:, :] = pad
    z_ref[N:, :] = pad


_scale = pl.pallas_call(
    _scale_body,
    out_shape=(
        jax.ShapeDtypeStruct((NP, D), jnp.float32),
        jax.ShapeDtypeStruct((NP, D), jnp.float32),
    ),
)


@functools.partial(
    pl.kernel,
    out_type=(
        jax.ShapeDtypeStruct((NP, D), jnp.float32),
        jax.ShapeDtypeStruct((NP, D), jnp.float32),
    ),
    mesh=_mesh,
    scratch_types=[
        pltpu.VMEM((IBLK, CHUNK), jnp.int32),
        pltpu.VMEM((IBLK, CHUNK), jnp.int32),
        pltpu.VMEM((KBUF, CHUNK, D), jnp.float32),
        pltpu.VMEM_SHARED((NP, D), jnp.float32),
        [pltpu.SemaphoreType.DMA] * KBUF,
        [pltpu.SemaphoreType.DMA] * KBUF,
    ],
)
def _agg_kernel(
    y_hbm, z_hbm, src_hbm, dst_hbm, aggsd_hbm, aggds_hbm,
    gidx_v, sidx_v, rows_v, acc_sh, sem_g, sem_s,
):
    c = lax.axis_index("c")
    s = lax.axis_index("s")

    zero16 = jnp.zeros((16,), jnp.float32)

    @pl.loop(0, CHUNK)
    def _(i):
        @pl.loop(0, D // 16)
        def _(j):
            rows_v[0, i, pl.ds(j * 16, 16)] = zero16

    @pl.loop(0, 7)
    def _(i):
        pltpu.sync_copy(rows_v.at[0], acc_sh.at[pl.ds(s * RPT + i * CHUNK, CHUNK)])

    pltpu.sync_copy(
        rows_v.at[0, pl.ds(0, 72)], acc_sh.at[pl.ds(s * RPT + 7 * CHUNK, 72)]
    )

    plsc.subcore_barrier()

    def run_dir(tab_hbm, gidx_hbm, sidx_hbm, out_hbm):
        def gather_start(j, b):
            pltpu.async_copy(tab_hbm.at[gidx_v.at[j]], rows_v.at[b], sem_g[b])

        def gather_wait(j, b):
            pltpu.make_async_copy(tab_hbm.at[gidx_v.at[j]], rows_v.at[b], sem_g[b]).wait()

        def scatter_start(j, b):
            pltpu.async_copy(rows_v.at[b], acc_sh.at[sidx_v.at[j]], sem_s[b], add=True)

        def scatter_wait(b):
            pltpu.make_async_copy(rows_v.at[b], acc_sh.at[sidx_v.at[0]], sem_s[b]).wait()

        @pl.loop(0, NIB)
        def _(p):
            pltpu.sync_copy(gidx_hbm.at[s, p], gidx_v)
            pltpu.sync_copy(sidx_hbm.at[s, p], sidx_v)

            @pl.loop(0, IBLK // KBUF)
            def _(t):
                j0 = t * KBUF
                for b in range(KBUF):
                    @pl.when(t > 0)
                    def _(b=b):
                        scatter_wait(b)

                    gather_start(j0 + b, b)
                for b in range(KBUF):
                    gather_wait(j0 + b, b)
                    scatter_start(j0 + b, b)

            for b in range(KBUF):
                scatter_wait(b)

        plsc.subcore_barrier()
        pltpu.sync_copy(
            acc_sh.at[pl.ds(s * RPT, RPT)], out_hbm.at[pl.ds(s * RPT, RPT)]
        )

    @pl.when(c == 0)
    def _():
        run_dir(y_hbm, src_hbm, dst_hbm, aggsd_hbm)

    @pl.when(c == 1)
    def _():
        run_dir(z_hbm, dst_hbm, src_hbm, aggds_hbm)


def _combine_body(
    asd_ref, ads_ref, dego_ref, degi_ref, wsd_ref, wds_ref, bsd_ref, bds_ref, o_ref
):
    dego = dego_ref[:N, 0:1]
    degi = degi_ref[:N, 0:1]
    dinv_out = jnp.where(dego > 0, lax.rsqrt(jnp.maximum(dego, 1e-12)), 0.0)
    dinv_in = jnp.where(degi > 0, lax.rsqrt(jnp.maximum(degi, 1e-12)), 0.0)
    asd = asd_ref[:N, :] * dinv_in
    ads = ads_ref[:N, :] * dinv_out
    t_sd = (
        jnp.dot(
            asd,
            wsd_ref[...],
            preferred_element_type=jnp.float32,
            precision=lax.Precision.HIGHEST,
        )
        + bsd_ref[...]
    )
    t_ds = (
        jnp.dot(
            ads,
            wds_ref[...],
            preferred_element_type=jnp.float32,
            precision=lax.Precision.HIGHEST,
        )
        + bds_ref[...]
    )
    o_ref[...] = ALPHA * t_sd + (1.0 - ALPHA) * t_ds


_combine = pl.pallas_call(
    _combine_body,
    out_shape=jax.ShapeDtypeStruct((N, D), jnp.float32),
)


def kernel(x, edge_index, W_sd, b_sd, W_ds, b_ds):
    src4 = edge_index[0].reshape(NS, NIB, IBLK, CHUNK)
    dst4 = edge_index[1].reshape(NS, NIB, IBLK, CHUNK)

    dego, degi = _deg_kernel(src4, dst4)
    y, z = _scale(x, dego, degi)
    aggsd, aggds = _agg_kernel(y, z, src4, dst4)
    return _combine(
        aggsd,
        aggds,
        dego,
        degi,
        W_sd,
        W_ds,
        b_sd.reshape(1, D),
        b_ds.reshape(1, D),
    )


# speedup vs baseline: 1.5994x; 1.0287x over previous
import functools

import jax
import jax.numpy as jnp
from jax import lax
from jax.experimental import pallas as pl
from jax.experimental.pallas import tpu as pltpu
from jax.experimental.pallas import tpu_sc as plsc

N = 10000
E = 320000
D = 128
ALPHA = 0.5

NC = 2
NS = 16
NP = 10112
RPT = NP // NS
EPW = E // NS
CHUNK = 100
KBUF = 2
IBLK = 50
NIB = 4
ZROWS = 8
KHIST = 8

_mesh = plsc.VectorSubcoreMesh(core_axis_name="c", subcore_axis_name="s")


@functools.partial(
    pl.kernel,
    out_type=(
        jax.ShapeDtypeStruct((NP, D), jnp.float32),
        jax.ShapeDtypeStruct((NP, D), jnp.float32),
    ),
    mesh=_mesh,
    scratch_types=[
        pltpu.VMEM((IBLK, CHUNK), jnp.int32),
        pltpu.VMEM((CHUNK, D), jnp.float32),
        pltpu.VMEM((ZROWS, D), jnp.float32),
        pltpu.VMEM_SHARED((NP, D), jnp.float32),
        pltpu.SemaphoreType.DMA,
    ],
)
def _deg_kernel(src_hbm, dst_hbm, dego_hbm, degi_hbm, idx_v, ones_v, zero_v, acc_sh, sem):
    c = lax.axis_index("c")
    s = lax.axis_index("s")

    one16 = jnp.full((16,), 1.0, jnp.float32)
    zero16 = jnp.zeros((16,), jnp.float32)

    @pl.loop(0, CHUNK)
    def _(i):
        @pl.loop(0, D // 16)
        def _(j):
            ones_v[i, pl.ds(j * 16, 16)] = one16

    @pl.loop(0, ZROWS)
    def _(i):
        @pl.loop(0, D // 16)
        def _(j):
            zero_v[i, pl.ds(j * 16, 16)] = zero16

    @pl.loop(0, RPT // ZROWS)
    def _(i):
        pltpu.sync_copy(zero_v, acc_sh.at[pl.ds(s * RPT + i * ZROWS, ZROWS)])

    plsc.subcore_barrier()

    def run_hist(idx_hbm, out_hbm):
        def drain_one():
            pltpu.make_async_copy(ones_v, acc_sh.at[idx_v.at[0]], sem).wait()

        @pl.loop(0, NIB)
        def _(p):
            pltpu.sync_copy(idx_hbm.at[s, p], idx_v)

            @pl.loop(0, IBLK)
            def _(j):
                pltpu.async_copy(ones_v, acc_sh.at[idx_v.at[j]], sem, add=True)

                @pl.when(j >= KHIST)
                def _():
                    drain_one()

            for _ in range(KHIST):
                drain_one()

        plsc.subcore_barrier()
        pltpu.sync_copy(
            acc_sh.at[pl.ds(s * RPT, RPT)], out_hbm.at[pl.ds(s * RPT, RPT)]
        )

    @pl.when(c == 0)
    def _():
        run_hist(src_hbm, dego_hbm)

    @pl.when(c == 1)
    def _():
        run_hist(dst_hbm, degi_hbm)


def _scale_body(x_ref, dego_ref, degi_ref, y_ref, z_ref):
    x = x_ref[...]
    dego = dego_ref[:N, 0:1]
    degi = degi_ref[:N, 0:1]
    dinv_out = jnp.where(dego > 0, lax.rsqrt(jnp.maximum(dego, 1e-12)), 0.0)
    dinv_in = jnp.where(degi > 0, lax.rsqrt(jnp.maximum(degi, 1e-12)), 0.0)
    y_ref[:N, :] = x * dinv_out
    z_ref[:N, :] = x * dinv_in
    pad = jnp.zeros((NP - N, D), jnp.float32)
    y_ref[N:, :] = pad
    z_ref[N:, :] = pad


_scale = pl.pallas_call(
    _scale_body,
    out_shape=(
        jax.ShapeDtypeStruct((NP, D), jnp.float32),
        jax.ShapeDtypeStruct((NP, D), jnp.float32),
    ),
)


@functools.partial(
    pl.kernel,
    out_type=(
        jax.ShapeDtypeStruct((NP, D), jnp.float32),
        jax.ShapeDtypeStruct((NP, D), jnp.float32),
    ),
    mesh=_mesh,
    scratch_types=[
        pltpu.VMEM((IBLK, CHUNK), jnp.int32),
        pltpu.VMEM((IBLK, CHUNK), jnp.int32),
        pltpu.VMEM((KBUF, CHUNK, D), jnp.float32),
        pltpu.VMEM_SHARED((NP, D), jnp.float32),
        [pltpu.SemaphoreType.DMA] * KBUF,
        [pltpu.SemaphoreType.DMA] * KBUF,
    ],
)
def _agg_kernel(
    y_hbm, z_hbm, src_hbm, dst_hbm, aggsd_hbm, aggds_hbm,
    gidx_v, sidx_v, rows_v, acc_sh, sem_g, sem_s,
):
    c = lax.axis_index("c")
    s = lax.axis_index("s")

    zero16 = jnp.zeros((16,), jnp.float32)

    @pl.loop(0, CHUNK)
    def _(i):
        @pl.loop(0, D // 16)
        def _(j):
            rows_v[0, i, pl.ds(j * 16, 16)] = zero16

    @pl.loop(0, 6)
    def _(i):
        pltpu.sync_copy(rows_v.at[0], acc_sh.at[pl.ds(s * RPT + i * CHUNK, CHUNK)])

    pltpu.sync_copy(
        rows_v.at[0, pl.ds(0, 32)], acc_sh.at[pl.ds(s * RPT + 6 * CHUNK, 32)]
    )

    plsc.subcore_barrier()

    def run_dir(tab_hbm, gidx_hbm, sidx_hbm, out_hbm):
        def gather_start(j, b):
            pltpu.async_copy(tab_hbm.at[gidx_v.at[j]], rows_v.at[b], sem_g[b])

        def gather_wait(j, b):
            pltpu.make_async_copy(tab_hbm.at[gidx_v.at[j]], rows_v.at[b], sem_g[b]).wait()

        def scatter_start(j, b):
            pltpu.async_copy(rows_v.at[b], acc_sh.at[sidx_v.at[j]], sem_s[b], add=True)

        def scatter_wait(b):
            pltpu.make_async_copy(rows_v.at[b], acc_sh.at[sidx_v.at[0]], sem_s[b]).wait()

        @pl.loop(0, NIB)
        def _(p):
            pltpu.sync_copy(gidx_hbm.at[s, p], gidx_v)
            pltpu.sync_copy(sidx_hbm.at[s, p], sidx_v)

            @pl.loop(0, IBLK // KBUF)
            def _(t):
                j0 = t * KBUF
                for b in range(KBUF):
                    @pl.when(t > 0)
                    def _(b=b):
                        scatter_wait(b)

                    gather_start(j0 + b, b)
                for b in range(KBUF):
                    gather_wait(j0 + b, b)
                    scatter_start(j0 + b, b)

            for b in range(KBUF):
                scatter_wait(b)

        plsc.subcore_barrier()
        pltpu.sync_copy(
            acc_sh.at[pl.ds(s * RPT, RPT)], out_hbm.at[pl.ds(s * RPT, RPT)]
        )

    @pl.when(c == 0)
    def _():
        run_dir(y_hbm, src_hbm, dst_hbm, aggsd_hbm)

    @pl.when(c == 1)
    def _():
        run_dir(z_hbm, dst_hbm, src_hbm, aggds_hbm)


def _combine_body(
    asd_ref, ads_ref, dego_ref, degi_ref, wsd_ref, wds_ref, bsd_ref, bds_ref, o_ref
):
    dego = dego_ref[:N, 0:1]
    degi = degi_ref[:N, 0:1]
    dinv_out = jnp.where(dego > 0, lax.rsqrt(jnp.maximum(dego, 1e-12)), 0.0)
    dinv_in = jnp.where(degi > 0, lax.rsqrt(jnp.maximum(degi, 1e-12)), 0.0)
    asd = asd_ref[:N, :] * dinv_in
    ads = ads_ref[:N, :] * dinv_out
    t_sd = (
        jnp.dot(
            asd,
            wsd_ref[...],
            preferred_element_type=jnp.float32,
            precision=lax.Precision.HIGHEST,
        )
        + bsd_ref[...]
    )
    t_ds = (
        jnp.dot(
            ads,
            wds_ref[...],
            preferred_element_type=jnp.float32,
            precision=lax.Precision.HIGHEST,
        )
        + bds_ref[...]
    )
    o_ref[...] = ALPHA * t_sd + (1.0 - ALPHA) * t_ds


_combine = pl.pallas_call(
    _combine_body,
    out_shape=jax.ShapeDtypeStruct((N, D), jnp.float32),
)


def kernel(x, edge_index, W_sd, b_sd, W_ds, b_ds):
    src4 = edge_index[0].reshape(NS, NIB, IBLK, CHUNK)
    dst4 = edge_index[1].reshape(NS, NIB, IBLK, CHUNK)

    dego, degi = _deg_kernel(src4, dst4)
    y, z = _scale(x, dego, degi)
    aggsd, aggds = _agg_kernel(y, z, src4, dst4)
    return _combine(
        aggsd,
        aggds,
        dego,
        degi,
        W_sd,
        W_ds,
        b_sd.reshape(1, D),
        b_ds.reshape(1, D),
    )


# speedup vs baseline: 1.6264x; 1.0169x over previous
import functools

import jax
import jax.numpy as jnp
from jax import lax
from jax.experimental import pallas as pl
from jax.experimental.pallas import tpu as pltpu
from jax.experimental.pallas import tpu_sc as plsc

N = 10000
E = 320000
D = 128
ALPHA = 0.5

NC = 2
NS = 16
NP = 10112
RPT = NP // NS
EPW = E // NS
CHUNK = 125
KBUF = 2
IBLK = 40
NIB = 4
ZROWS = 8
KHIST = 8

_mesh = plsc.VectorSubcoreMesh(core_axis_name="c", subcore_axis_name="s")


@functools.partial(
    pl.kernel,
    out_type=(
        jax.ShapeDtypeStruct((NP, D), jnp.float32),
        jax.ShapeDtypeStruct((NP, D), jnp.float32),
    ),
    mesh=_mesh,
    scratch_types=[
        pltpu.VMEM((IBLK, CHUNK), jnp.int32),
        pltpu.VMEM((CHUNK, D), jnp.float32),
        pltpu.VMEM((ZROWS, D), jnp.float32),
        pltpu.VMEM_SHARED((NP, D), jnp.float32),
        pltpu.SemaphoreType.DMA,
    ],
)
def _deg_kernel(src_hbm, dst_hbm, dego_hbm, degi_hbm, idx_v, ones_v, zero_v, acc_sh, sem):
    c = lax.axis_index("c")
    s = lax.axis_index("s")

    one16 = jnp.full((16,), 1.0, jnp.float32)
    zero16 = jnp.zeros((16,), jnp.float32)

    @pl.loop(0, CHUNK)
    def _(i):
        @pl.loop(0, D // 16)
        def _(j):
            ones_v[i, pl.ds(j * 16, 16)] = one16

    @pl.loop(0, ZROWS)
    def _(i):
        @pl.loop(0, D // 16)
        def _(j):
            zero_v[i, pl.ds(j * 16, 16)] = zero16

    @pl.loop(0, RPT // ZROWS)
    def _(i):
        pltpu.sync_copy(zero_v, acc_sh.at[pl.ds(s * RPT + i * ZROWS, ZROWS)])

    plsc.subcore_barrier()

    def run_hist(idx_hbm, out_hbm):
        def drain_one():
            pltpu.make_async_copy(ones_v, acc_sh.at[idx_v.at[0]], sem).wait()

        @pl.loop(0, NIB)
        def _(p):
            pltpu.sync_copy(idx_hbm.at[s, p], idx_v)

            @pl.loop(0, IBLK)
            def _(j):
                pltpu.async_copy(ones_v, acc_sh.at[idx_v.at[j]], sem, add=True)

                @pl.when(j >= KHIST)
                def _():
                    drain_one()

            for _ in range(KHIST):
                drain_one()

        plsc.subcore_barrier()
        pltpu.sync_copy(
            acc_sh.at[pl.ds(s * RPT, RPT)], out_hbm.at[pl.ds(s * RPT, RPT)]
        )

    @pl.when(c == 0)
    def _():
        run_hist(src_hbm, dego_hbm)

    @pl.when(c == 1)
    def _():
        run_hist(dst_hbm, degi_hbm)


def _scale_body(x_ref, dego_ref, degi_ref, y_ref, z_ref):
    x = x_ref[...]
    dego = dego_ref[:N, 0:1]
    degi = degi_ref[:N, 0:1]
    dinv_out = jnp.where(dego > 0, lax.rsqrt(jnp.maximum(dego, 1e-12)), 0.0)
    dinv_in = jnp.where(degi > 0, lax.rsqrt(jnp.maximum(degi, 1e-12)), 0.0)
    y_ref[:N, :] = x * dinv_out
    z_ref[:N, :] = x * dinv_in
    pad = jnp.zeros((NP - N, D), jnp.float32)
    y_ref[N:, :] = pad
    z_ref[N:, :] = pad


_scale = pl.pallas_call(
    _scale_body,
    out_shape=(
        jax.ShapeDtypeStruct((NP, D), jnp.float32),
        jax.ShapeDtypeStruct((NP, D), jnp.float32),
    ),
)


@functools.partial(
    pl.kernel,
    out_type=(
        jax.ShapeDtypeStruct((NP, D), jnp.float32),
        jax.ShapeDtypeStruct((NP, D), jnp.float32),
    ),
    mesh=_mesh,
    scratch_types=[
        pltpu.VMEM((IBLK, CHUNK), jnp.int32),
        pltpu.VMEM((IBLK, CHUNK), jnp.int32),
        pltpu.VMEM((KBUF, CHUNK, D), jnp.float32),
        pltpu.VMEM_SHARED((NP, D), jnp.float32),
        [pltpu.SemaphoreType.DMA] * KBUF,
        [pltpu.SemaphoreType.DMA] * KBUF,
    ],
)
def _agg_kernel(
    y_hbm, z_hbm, src_hbm, dst_hbm, aggsd_hbm, aggds_hbm,
    gidx_v, sidx_v, rows_v, acc_sh, sem_g, sem_s,
):
    c = lax.axis_index("c")
    s = lax.axis_index("s")

    zero16 = jnp.zeros((16,), jnp.float32)

    @pl.loop(0, CHUNK)
    def _(i):
        @pl.loop(0, D // 16)
        def _(j):
            rows_v[0, i, pl.ds(j * 16, 16)] = zero16

    @pl.loop(0, 5)
    def _(i):
        pltpu.sync_copy(
            rows_v.at[0, pl.ds(0, 120)], acc_sh.at[pl.ds(s * RPT + i * 120, 120)]
        )

    pltpu.sync_copy(
        rows_v.at[0, pl.ds(0, 32)], acc_sh.at[pl.ds(s * RPT + 600, 32)]
    )

    plsc.subcore_barrier()

    def run_dir(tab_hbm, gidx_hbm, sidx_hbm, out_hbm):
        def gather_start(j, b):
            pltpu.async_copy(tab_hbm.at[gidx_v.at[j]], rows_v.at[b], sem_g[b])

        def gather_wait(j, b):
            pltpu.make_async_copy(tab_hbm.at[gidx_v.at[j]], rows_v.at[b], sem_g[b]).wait()

        def scatter_start(j, b):
            pltpu.async_copy(rows_v.at[b], acc_sh.at[sidx_v.at[j]], sem_s[b], add=True)

        def scatter_wait(b):
            pltpu.make_async_copy(rows_v.at[b], acc_sh.at[sidx_v.at[0]], sem_s[b]).wait()

        @pl.loop(0, NIB)
        def _(p):
            pltpu.sync_copy(gidx_hbm.at[s, p], gidx_v)
            pltpu.sync_copy(sidx_hbm.at[s, p], sidx_v)

            @pl.loop(0, IBLK // KBUF)
            def _(t):
                j0 = t * KBUF
                for b in range(KBUF):
                    @pl.when(t > 0)
                    def _(b=b):
                        scatter_wait(b)

                    gather_start(j0 + b, b)
                for b in range(KBUF):
                    gather_wait(j0 + b, b)
                    scatter_start(j0 + b, b)

            for b in range(KBUF):
                scatter_wait(b)

        plsc.subcore_barrier()
        pltpu.sync_copy(
            acc_sh.at[pl.ds(s * RPT, RPT)], out_hbm.at[pl.ds(s * RPT, RPT)]
        )

    @pl.when(c == 0)
    def _():
        run_dir(y_hbm, src_hbm, dst_hbm, aggsd_hbm)

    @pl.when(c == 1)
    def _():
        run_dir(z_hbm, dst_hbm, src_hbm, aggds_hbm)


def _combine_body(
    asd_ref, ads_ref, dego_ref, degi_ref, wsd_ref, wds_ref, bsd_ref, bds_ref, o_ref
):
    dego = dego_ref[:N, 0:1]
    degi = degi_ref[:N, 0:1]
    dinv_out = jnp.where(dego > 0, lax.rsqrt(jnp.maximum(dego, 1e-12)), 0.0)
    dinv_in = jnp.where(degi > 0, lax.rsqrt(jnp.maximum(degi, 1e-12)), 0.0)
    asd = asd_ref[:N, :] * dinv_in
    ads = ads_ref[:N, :] * dinv_out
    t_sd = (
        jnp.dot(
            asd,
            wsd_ref[...],
            preferred_element_type=jnp.float32,
            precision=lax.Precision.HIGHEST,
        )
        + bsd_ref[...]
    )
    t_ds = (
        jnp.dot(
            ads,
            wds_ref[...],
            preferred_element_type=jnp.float32,
            precision=lax.Precision.HIGHEST,
        )
        + bds_ref[...]
    )
    o_ref[...] = ALPHA * t_sd + (1.0 - ALPHA) * t_ds


_combine = pl.pallas_call(
    _combine_body,
    out_shape=jax.ShapeDtypeStruct((N, D), jnp.float32),
)


def kernel(x, edge_index, W_sd, b_sd, W_ds, b_ds):
    src4 = edge_index[0].reshape(NS, NIB, IBLK, CHUNK)
    dst4 = edge_index[1].reshape(NS, NIB, IBLK, CHUNK)

    dego, degi = _deg_kernel(src4, dst4)
    y, z = _scale(x, dego, degi)
    aggsd, aggds = _agg_kernel(y, z, src4, dst4)
    return _combine(
        aggsd,
        aggds,
        dego,
        degi,
        W_sd,
        W_ds,
        b_sd.reshape(1, D),
        b_ds.reshape(1, D),
    )
